# baseline (speedup 1.0000x reference)
import functools

import jax
import jax.numpy as jnp
from jax import lax
from jax.experimental import pallas as pl
from jax.experimental.pallas import tpu as pltpu
from jax.experimental.pallas import tpu_sc as plsc

BATCH = 16384
FACTORS = 64
NUM_WORKERS = 32
ROWS_PER_W = BATCH // NUM_WORKERS
CHUNK = 128
NCHUNK = ROWS_PER_W // CHUNK
GROUPS_PER_CHUNK = CHUNK // 16
PAIR_ROWS = 1000000 // 2
PAIR_F = 2 * FACTORS
KB = FACTORS // 16

_mesh = plsc.VectorSubcoreMesh(core_axis_name="c", subcore_axis_name="s")


def _worker_id():
    return lax.axis_index("s") * 2 + lax.axis_index("c")


def _parity_splat(par, s):
    lane = jnp.full((16,), s, jnp.int32)
    return par.at[lane].get(mode="promise_in_bounds") == 1


@functools.partial(
    pl.kernel,
    mesh=_mesh,
    out_type=jax.ShapeDtypeStruct((BATCH, FACTORS), jnp.float32),
    scratch_types=[
        pltpu.VMEM((NCHUNK, CHUNK), jnp.int32),
        pltpu.VMEM((NCHUNK, CHUNK), jnp.int32),
        pltpu.VMEM((2, CHUNK, PAIR_F), jnp.float32),
        pltpu.VMEM((CHUNK, FACTORS), jnp.float32),
        pltpu.SemaphoreType.DMA,
        pltpu.SemaphoreType.DMA,
    ],
    compiler_params=pltpu.CompilerParams(needs_layout_passes=False),
)
def _gather_sc(idx_hbm, tab_hbm, out_hbm, ix_v, px_v, rows_v, sel_v,
               sem0, sem1):
    sems = (sem0, sem1)
    wid = _worker_id()

    pltpu.sync_copy(idx_hbm.at[wid], ix_v)
    for c in range(NCHUNK):
        for k in range(CHUNK // 16):
            sl = pl.ds(k * 16, 16)
            px_v[c, sl] = lax.shift_right_logical(ix_v[c, sl], 1)

    def start_chunk(c):
        return pltpu.async_copy(tab_hbm.at[px_v.at[c]], rows_v.at[c % 2],
                                sems[c % 2])

    inflight = {0: start_chunk(0)}
    for c in range(NCHUNK):
        if c + 1 < NCHUNK:
            inflight[c + 1] = start_chunk(c + 1)
        inflight.pop(c).wait()
        rows_c = rows_v.at[c % 2]

        def group_body(g, carry, c=c, rows_c=rows_c):
            par = ix_v[c, pl.ds(g * 16, 16)] & 1
            for s in range(16):
                r = g * 16 + s
                ps = _parity_splat(par, s)
                for k in range(KB):
                    lo = rows_c[r, pl.ds(k * 16, 16)]
                    hi = rows_c[r, pl.ds(FACTORS + k * 16, 16)]
                    sel_v[r, pl.ds(k * 16, 16)] = jnp.where(ps, hi, lo)
            return carry

        lax.fori_loop(0, GROUPS_PER_CHUNK, group_body, 0)
        pltpu.sync_copy(
            sel_v, out_hbm.at[pl.ds(wid * ROWS_PER_W + c * CHUNK, CHUNK)])


@functools.partial(
    pl.kernel,
    mesh=_mesh,
    out_type=jax.ShapeDtypeStruct((BATCH,), jnp.float32),
    scratch_types=[
        pltpu.VMEM((NCHUNK, CHUNK), jnp.int32),
        pltpu.VMEM((NCHUNK, CHUNK), jnp.int32),
        pltpu.VMEM((2, CHUNK, PAIR_F), jnp.float32),
        pltpu.VMEM((2, CHUNK, FACTORS), jnp.float32),
        pltpu.VMEM((FACTORS,), jnp.float32),
        pltpu.VMEM((16,), jnp.float32),
        pltpu.VMEM((ROWS_PER_W,), jnp.float32),
        pltpu.SemaphoreType.DMA,
        pltpu.SemaphoreType.DMA,
    ],
    compiler_params=pltpu.CompilerParams(needs_layout_passes=False),
)
def _dot_sc(iidx_hbm, it_hbm, us_hbm, w_hbm, b_hbm, out_hbm,
            ix_v, px_v, rows_v, us_v, w_v, b_v, out_v, sem0, sem1):
    sems = (sem0, sem1)
    wid = _worker_id()

    pltpu.sync_copy(iidx_hbm.at[wid], ix_v)
    pltpu.sync_copy(w_hbm, w_v)
    pltpu.sync_copy(b_hbm, b_v)
    for c in range(NCHUNK):
        for k in range(CHUNK // 16):
            sl = pl.ds(k * 16, 16)
            px_v[c, sl] = lax.shift_right_logical(ix_v[c, sl], 1)

    def start_chunk(c):
        buf = c % 2
        return (
            pltpu.async_copy(it_hbm.at[px_v.at[c]], rows_v.at[buf],
                             sems[buf]),
            pltpu.async_copy(
                us_hbm.at[pl.ds(wid * ROWS_PER_W + c * CHUNK, CHUNK)],
                us_v.at[buf], sems[buf]),
        )

    iota16 = lax.iota(jnp.int32, 16)
    last_lane = iota16 == 15
    b_onehot = jnp.where(iota16 == 0, b_v[...], 0.0)
    wv = [w_v[pl.ds(k * 16, 16)] for k in range(KB)]

    inflight = {0: start_chunk(0)}
    for c in range(NCHUNK):
        if c + 1 < NCHUNK:
            inflight[c + 1] = start_chunk(c + 1)
        for cp in inflight.pop(c):
            cp.wait()
        buf = c % 2
        rows_c = rows_v.at[buf]
        us_c = us_v.at[buf]

        def group_body(g, carry, c=c, rows_c=rows_c, us_c=us_c):
            par = ix_v[c, pl.ds(g * 16, 16)] & 1
            for s in range(16):
                r = g * 16 + s
                ps = _parity_splat(par, s)
                acc = b_onehot
                for k in range(KB):
                    lo = rows_c[r, pl.ds(k * 16, 16)]
                    hi = rows_c[r, pl.ds(FACTORS + k * 16, 16)]
                    v = jnp.where(ps, hi, lo)
                    acc = acc + us_c[r, pl.ds(k * 16, 16)] * v * wv[k]
                tot = plsc.cumsum(acc)
                plsc.store_scatter(
                    out_v, [jnp.full((16,), c * CHUNK, jnp.int32) + r],
                    tot, mask=last_lane)
            return carry

        lax.fori_loop(0, GROUPS_PER_CHUNK, group_body, 0)

    pltpu.sync_copy(out_v, out_hbm.at[pl.ds(wid * ROWS_PER_W, ROWS_PER_W)])


def kernel(user_idx, item_idx, user_table, item_table, W, b):
    uidx = user_idx.reshape(NUM_WORKERS, NCHUNK, CHUNK)
    iidx = item_idx.reshape(NUM_WORKERS, NCHUNK, CHUNK)
    ut2 = user_table.reshape(PAIR_ROWS, PAIR_F)
    it2 = item_table.reshape(PAIR_ROWS, PAIR_F)
    w = W.reshape(FACTORS)
    bvec = jnp.broadcast_to(b, (16,)).astype(jnp.float32)
    u_sel = _gather_sc(uidx, ut2)
    return _dot_sc(iidx, it2, u_sel, w, bvec)
